# Initial kernel scaffold; baseline (speedup 1.0000x reference)
import jax
import jax.numpy as jnp
from jax.experimental import pallas as pl


def kernel(feature_ids, feature_ratings, item_ids, feature_table, rating_table, item_table):
    raise NotImplementedError("write your pallas kernel here")


# speedup vs baseline: 9.5970x; 9.5970x over previous
import functools

import jax
import jax.numpy as jnp
from jax import lax
from jax.experimental import pallas as pl
from jax.experimental.pallas import tpu as pltpu
from jax.experimental.pallas import tpu_sc as plsc

B = 4096
L = 50
D = 64
NR = 10

_info = plsc.get_sparse_core_info()
NC = _info.num_cores
NS = _info.num_subcores
LANES = _info.num_lanes
NW = NC * NS
UPW = B // NW
UC = 8
ROWS = UC * L
GSUB = 80
NG = ROWS // GSUB

_mesh = plsc.VectorSubcoreMesh(core_axis_name="c", subcore_axis_name="s")


@functools.partial(
    pl.kernel,
    mesh=_mesh,
    compiler_params=pltpu.CompilerParams(
        needs_layout_passes=False, use_tc_tiling_on_sc=False),
    out_type=(
        jax.ShapeDtypeStruct((B, D), jnp.float32),
        jax.ShapeDtypeStruct((B, D), jnp.float32),
    ),
    scratch_types=[
        pltpu.VMEM((UPW * L,), jnp.int32),
        pltpu.VMEM((UPW * L,), jnp.float32),
        pltpu.VMEM((UPW * L,), jnp.int32),
        pltpu.VMEM((NR * D,), jnp.float32),
        pltpu.VMEM((ROWS, D), jnp.float32),
        pltpu.VMEM((UPW,), jnp.int32),
        pltpu.VMEM((UPW, D), jnp.float32),
        pltpu.VMEM((UC, D), jnp.float32),
        pltpu.SemaphoreType.DMA,
    ],
)
def _emb_kernel(fid_hbm, rat_hbm, iid_hbm, ftab_hbm, rt_hbm, itab_hbm,
                user_out, item_out,
                fid_v, rat_v, ridx_v, rt_v, rows_v, iid_v, item_rows, acc_v,
                sem):
  wid = lax.axis_index("s") * NC + lax.axis_index("c")
  ubase = pl.multiple_of(wid * UPW, UPW)
  rbase = pl.multiple_of(wid * (UPW * L), 8)

  pltpu.sync_copy(fid_hbm.at[pl.ds(rbase, UPW * L)], fid_v)
  pltpu.sync_copy(rat_hbm.at[pl.ds(rbase, UPW * L)], rat_v)
  pltpu.sync_copy(rt_hbm, rt_v)
  pltpu.sync_copy(iid_hbm.at[pl.ds(ubase, UPW)], iid_v)

  pltpu.async_copy(itab_hbm.at[iid_v], item_rows, sem).wait()
  pltpu.sync_copy(item_rows, item_out.at[pl.ds(ubase, UPW)])

  def _ridx(i, carry):
    r = rat_v[pl.ds(i * LANES, LANES)]
    ridx_v[pl.ds(i * LANES, LANES)] = jnp.clip(
        ((r - 0.5) * 2.0).astype(jnp.int32), 0, 9)
    return carry
  lax.fori_loop(0, UPW * L // LANES, _ridx, 0, unroll=4)

  offs = [lax.iota(jnp.int32, LANES) + dg * LANES for dg in range(4)]
  zeros = jnp.zeros((LANES,), jnp.float32)

  def _chunk(c, carry):
    crow = pl.multiple_of(c * ROWS, 8)
    cps = []
    for k in range(NG):
      cps.append(pltpu.async_copy(
          ftab_hbm.at[fid_v.at[pl.ds(crow + k * GSUB, GSUB)]],
          rows_v.at[pl.ds(k * GSUB, GSUB)], sem))
    for cp in cps:
      cp.wait()
    for u in range(UC):
      def _l(l, accs, u=u):
        rbc = plsc.load_gather(
            ridx_v, [jnp.full((LANES,), crow + u * L + l, jnp.int32)])
        base = rbc * D
        row = u * L + l
        new = []
        for dg in range(4):
          rtv = plsc.load_gather(rt_v, [base + offs[dg]])
          fv = rows_v[row, pl.ds(dg * LANES, LANES)]
          new.append(accs[dg] + fv * rtv)
        return tuple(new)
      accs = lax.fori_loop(0, L, _l, (zeros, zeros, zeros, zeros))
      for dg in range(4):
        acc_v[u, pl.ds(dg * LANES, LANES)] = accs[dg]
    pltpu.sync_copy(acc_v, user_out.at[pl.ds(ubase + c * UC, UC)])
    return carry
  lax.fori_loop(0, UPW // UC, _chunk, 0)


def kernel(feature_ids, feature_ratings, item_ids, feature_table,
           rating_table, item_table):
  fid = feature_ids.reshape(-1).astype(jnp.int32)
  rat = feature_ratings.reshape(-1)
  iid = item_ids.astype(jnp.int32)
  rt = rating_table.reshape(-1)
  user, item = _emb_kernel(fid, rat, iid, feature_table, rt, item_table)
  return jnp.stack((user, item), axis=1)
